# baseline (speedup 1.0000x reference)
import functools

import jax
import jax.numpy as jnp
import numpy as np
from jax import lax
from jax.experimental import pallas as pl
from jax.experimental.pallas import tpu as pltpu
from jax.experimental.pallas import tpu_sc as plsc

F = 26
V = 100000
D = 16
B = 4096
EPS = 1e-5
NPAIR = F * (F - 1) // 2
_ROWS_NP, _COLS_NP = np.triu_indices(F, k=1)

_NC = 2
_NS = 16
_NW = _NC * _NS
_TOT = B * F
_PER_W = _TOT // _NW


def _sc_gather(emb_flat, lin_blk, idx, blk):
    mesh = plsc.VectorSubcoreMesh(core_axis_name="c", subcore_axis_name="s")

    @functools.partial(
        pl.kernel,
        mesh=mesh,
        compiler_params=pltpu.CompilerParams(
            use_tc_tiling_on_sc=False, needs_layout_passes=False),
        out_type=[
            jax.ShapeDtypeStruct((_TOT, D), jnp.float32),
            jax.ShapeDtypeStruct((_TOT,), jnp.float32),
        ],
        scratch_types=[
            pltpu.VMEM((_PER_W,), jnp.int32),
            pltpu.VMEM((_PER_W,), jnp.int32),
            pltpu.VMEM((_PER_W, D), jnp.float32),
            pltpu.VMEM((_PER_W, 16), jnp.float32),
            pltpu.VMEM((_PER_W,), jnp.float32),
            pltpu.SemaphoreType.DMA,
            pltpu.SemaphoreType.DMA,
        ],
    )
    def gather_kernel(emb_hbm, linb_hbm, idx_hbm, blk_hbm, e_out, l_out,
                      idx_v, blk_v, rows_v, linb_v, lin_v, sem_e, sem_l):
        wid = lax.axis_index("s") * _NC + lax.axis_index("c")
        base = wid * _PER_W
        pltpu.sync_copy(idx_hbm.at[pl.ds(base, _PER_W)], idx_v)
        pltpu.sync_copy(blk_hbm.at[pl.ds(base, _PER_W)], blk_v)
        ce = pltpu.async_copy(emb_hbm.at[idx_v], rows_v, sem_e)
        cl = pltpu.async_copy(linb_hbm.at[blk_v], linb_v, sem_l)
        ce.wait()
        cl.wait()

        lane_iota = lax.iota(jnp.int32, 16)

        def lane_step(c, carry):
            b16 = c * 16
            lanes = idx_v[pl.ds(b16, 16)] & 15
            rows16 = b16 + lane_iota
            lin_v[pl.ds(b16, 16)] = plsc.load_gather(linb_v, [rows16, lanes])
            return carry

        lax.fori_loop(0, _PER_W // 16, lane_step, 0)
        pltpu.sync_copy(rows_v, e_out.at[pl.ds(base, _PER_W)])
        pltpu.sync_copy(lin_v, l_out.at[pl.ds(base, _PER_W)])

    return gather_kernel(emb_flat, lin_blk, idx, blk)


def _dense_body(rows_ref, cols_ref, eT_ref, linT_ref, alpha_ref, beta_ref,
                out_ref, inner_ref):
    def pair_step(p, carry):
        i = rows_ref[p]
        j = cols_ref[p]
        a = eT_ref[pl.ds(pl.multiple_of(i * D, D), D), :]
        b = eT_ref[pl.ds(pl.multiple_of(j * D, D), D), :]
        inner_ref[pl.ds(p, 1), :] = jnp.sum(a * b, axis=0, keepdims=True)
        return carry

    lax.fori_loop(0, NPAIR, pair_step, 0)

    inner = inner_ref[:, :]
    m = jnp.mean(inner, axis=1, keepdims=True)
    ex2 = jnp.mean(inner * inner, axis=1, keepdims=True)
    w = jnp.tanh(beta_ref[:, :]) * lax.rsqrt(ex2 - m * m + EPS)
    fm = jnp.sum(w * (inner - m), axis=0, keepdims=True)

    lin = linT_ref[:, :]
    lm = jnp.mean(lin, axis=1, keepdims=True)
    lex2 = jnp.mean(lin * lin, axis=1, keepdims=True)
    la = jnp.tanh(alpha_ref[:, :]) * lax.rsqrt(lex2 - lm * lm + EPS)
    lout = jnp.sum(la * (lin - lm), axis=0, keepdims=True)

    out_ref[:, :] = jax.nn.sigmoid(lout + fm)


def _dense(eT, linT, alpha_col, beta_col, rows, cols):
    return pl.pallas_call(
        _dense_body,
        out_shape=jax.ShapeDtypeStruct((1, B), jnp.float32),
        in_specs=[
            pl.BlockSpec(memory_space=pltpu.SMEM),
            pl.BlockSpec(memory_space=pltpu.SMEM),
            pl.BlockSpec(memory_space=pltpu.VMEM),
            pl.BlockSpec(memory_space=pltpu.VMEM),
            pl.BlockSpec(memory_space=pltpu.VMEM),
            pl.BlockSpec(memory_space=pltpu.VMEM),
        ],
        out_specs=pl.BlockSpec(memory_space=pltpu.VMEM),
        scratch_shapes=[pltpu.VMEM((NPAIR, B), jnp.float32)],
    )(rows, cols, eT, linT, alpha_col, beta_col)


def kernel(x, lin_w, emb, alpha, beta):
    idx = (x.astype(jnp.int32)
           + (jnp.arange(F, dtype=jnp.int32) * V)[None, :]).reshape(-1)
    e_flat, l_flat = _sc_gather(
        emb.reshape(F * V, D), lin_w.reshape(F * V // 16, 16), idx,
        jax.lax.shift_right_logical(idx, 4))
    eT = e_flat.reshape(B, F, D).transpose(1, 2, 0).reshape(F * D, B)
    linT = l_flat.reshape(B, F).T
    rows = jnp.asarray(_ROWS_NP, dtype=jnp.int32)
    cols = jnp.asarray(_COLS_NP, dtype=jnp.int32)
    out = _dense(eT, linT, alpha.reshape(F, 1), beta.reshape(NPAIR, 1),
                 rows, cols)
    return out.reshape(B, 1)


# speedup vs baseline: 1.0913x; 1.0913x over previous
import functools

import jax
import jax.numpy as jnp
import numpy as np
from jax import lax
from jax.experimental import pallas as pl
from jax.experimental.pallas import tpu as pltpu
from jax.experimental.pallas import tpu_sc as plsc

F = 26
V = 100000
D = 16
B = 4096
EPS = 1e-5
NPAIR = F * (F - 1) // 2
_ROWS_NP, _COLS_NP = np.triu_indices(F, k=1)

_NC = 2
_NS = 16
_NW = _NC * _NS
_TOT = B * F
_PER_W = _TOT // _NW


def _sc_gather(emb_flat, lin_blk, idx, blk):
    mesh = plsc.VectorSubcoreMesh(core_axis_name="c", subcore_axis_name="s")

    @functools.partial(
        pl.kernel,
        mesh=mesh,
        compiler_params=pltpu.CompilerParams(
            use_tc_tiling_on_sc=False, needs_layout_passes=False),
        out_type=[
            jax.ShapeDtypeStruct((_TOT, D), jnp.float32),
            jax.ShapeDtypeStruct((_TOT,), jnp.float32),
        ],
        scratch_types=[
            pltpu.VMEM((_PER_W,), jnp.int32),
            pltpu.VMEM((_PER_W,), jnp.int32),
            pltpu.VMEM((_PER_W, D), jnp.float32),
            pltpu.VMEM((_PER_W, 16), jnp.float32),
            pltpu.VMEM((_PER_W,), jnp.float32),
            pltpu.SemaphoreType.DMA,
            pltpu.SemaphoreType.DMA,
        ],
    )
    def gather_kernel(emb_hbm, linb_hbm, idx_hbm, blk_hbm, e_out, l_out,
                      idx_v, blk_v, rows_v, linb_v, lin_v, sem_e, sem_l):
        wid = lax.axis_index("s") * _NC + lax.axis_index("c")
        base = wid * _PER_W
        pltpu.sync_copy(idx_hbm.at[pl.ds(base, _PER_W)], idx_v)
        pltpu.sync_copy(blk_hbm.at[pl.ds(base, _PER_W)], blk_v)
        ce = pltpu.async_copy(emb_hbm.at[idx_v], rows_v, sem_e)
        cl = pltpu.async_copy(linb_hbm.at[blk_v], linb_v, sem_l)
        ce.wait()
        cl.wait()

        lane_iota = lax.iota(jnp.int32, 16)

        def lane_step(c, carry):
            b16 = c * 16
            lanes = idx_v[pl.ds(b16, 16)] & 15
            rows16 = b16 + lane_iota
            lin_v[pl.ds(b16, 16)] = plsc.load_gather(linb_v, [rows16, lanes])
            return carry

        lax.fori_loop(0, _PER_W // 16, lane_step, 0)
        pltpu.sync_copy(rows_v, e_out.at[pl.ds(base, _PER_W)])
        pltpu.sync_copy(lin_v, l_out.at[pl.ds(base, _PER_W)])

    return gather_kernel(emb_flat, lin_blk, idx, blk)


_BBLK = 128
_NBLK = B // _BBLK


def _dense_body(e2_ref, lin2_ref, alpha_ref, beta_ref, out_ref,
                innerT_s, linT_s):
    p = pl.program_id(0)
    lane0 = pl.multiple_of(p * _BBLK, _BBLK)

    eT_blk = lax.transpose(e2_ref[:, :], (1, 0))
    lin_t = lax.transpose(lin2_ref[:, :], (1, 0))
    linT_s[:, pl.ds(lane0, _BBLK)] = lin_t

    svals = []
    for q in range(NPAIR):
        i = int(_ROWS_NP[q]) * D
        j = int(_COLS_NP[q]) * D
        prod = eT_blk[i:i + D, :] * eT_blk[j:j + D, :]
        svals.append(jnp.sum(prod, axis=0, keepdims=True))
    innerT_s[:, pl.ds(lane0, _BBLK)] = jnp.concatenate(svals, axis=0)

    @pl.when(p == _NBLK - 1)
    def _stats():
        inner = innerT_s[:, :]
        m = jnp.mean(inner, axis=1, keepdims=True)
        ex2 = jnp.mean(inner * inner, axis=1, keepdims=True)
        w = jnp.tanh(beta_ref[:, :]) * lax.rsqrt(ex2 - m * m + EPS)
        fm = jnp.sum(w * (inner - m), axis=0, keepdims=True)

        lin = linT_s[:, :]
        lm = jnp.mean(lin, axis=1, keepdims=True)
        lex2 = jnp.mean(lin * lin, axis=1, keepdims=True)
        la = jnp.tanh(alpha_ref[:, :]) * lax.rsqrt(lex2 - lm * lm + EPS)
        lout = jnp.sum(la * (lin - lm), axis=0, keepdims=True)

        out_ref[:, :] = lax.transpose(jax.nn.sigmoid(lout + fm), (1, 0))


def _dense(e2, lin2, alpha_col, beta_col):
    return pl.pallas_call(
        _dense_body,
        grid=(_NBLK,),
        out_shape=jax.ShapeDtypeStruct((B, 1), jnp.float32),
        in_specs=[
            pl.BlockSpec((_BBLK, F * D), lambda p: (p, 0)),
            pl.BlockSpec((_BBLK, F), lambda p: (p, 0)),
            pl.BlockSpec((F, 1), lambda p: (0, 0)),
            pl.BlockSpec((NPAIR, 1), lambda p: (0, 0)),
        ],
        out_specs=pl.BlockSpec((B, 1), lambda p: (0, 0)),
        scratch_shapes=[
            pltpu.VMEM((NPAIR, B), jnp.float32),
            pltpu.VMEM((F, B), jnp.float32),
        ],
    )(e2, lin2, alpha_col, beta_col)


def kernel(x, lin_w, emb, alpha, beta):
    idx = (x.astype(jnp.int32)
           + (jnp.arange(F, dtype=jnp.int32) * V)[None, :]).reshape(-1)
    e_flat, l_flat = _sc_gather(
        emb.reshape(F * V, D), lin_w.reshape(F * V // 16, 16), idx,
        jax.lax.shift_right_logical(idx, 4))
    e2 = e_flat.reshape(B, F * D)
    lin2 = l_flat.reshape(B, F)
    return _dense(e2, lin2, alpha.reshape(F, 1), beta.reshape(NPAIR, 1))
